# Initial kernel scaffold; baseline (speedup 1.0000x reference)
import jax
import jax.numpy as jnp
from jax.experimental import pallas as pl


def kernel(adj_rows, adj_cols, adj_vals, input_feature, weights):
    raise NotImplementedError("write your pallas kernel here")


# speedup vs baseline: 3.4364x; 3.4364x over previous
import functools

import jax
import jax.numpy as jnp
from jax import lax
from jax.experimental import pallas as pl
from jax.experimental.pallas import tpu as pltpu
from jax.experimental.pallas import tpu_sc as plsc

N = 10000
E = 160000
F = 256
FH = 128
NC = 2
NS = 16
B = 128
NB = -(-E // (NS * B))
E_PAD = NS * NB * B
N_PAD = 10240
RPT = N_PAD // NS
GROUPS = B // 16
FV = FH // 16


def _matmul_body(w_ref, x_ref, o_ref):
    o_ref[...] = jnp.dot(w_ref[...], x_ref[...],
                         preferred_element_type=jnp.float32)


def _support_halves(weights, input_feature):
    return pl.pallas_call(
        _matmul_body,
        grid=(NC, 25),
        in_specs=[
            pl.BlockSpec((400, F), lambda c, i: (i, 0)),
            pl.BlockSpec((F, FH), lambda c, i: (0, c)),
        ],
        out_specs=pl.BlockSpec((400, FH), lambda c, i: (c * 25 + i, 0)),
        out_shape=jax.ShapeDtypeStruct((NC * N, FH), jnp.float32),
    )(weights, input_feature)


def _splat_lane(v, lane):
    idx = jnp.full((16,), lane, dtype=jnp.int32)
    return lax.gather(
        v, idx[:, None],
        dimension_numbers=lax.GatherDimensionNumbers(
            offset_dims=(), collapsed_slice_dims=(0,), start_index_map=(0,)),
        slice_sizes=(1,),
        mode=lax.GatherScatterMode.PROMISE_IN_BOUNDS)


_MESH = plsc.VectorSubcoreMesh(core_axis_name="c", subcore_axis_name="s")


@functools.partial(
    pl.kernel,
    out_type=jax.ShapeDtypeStruct((NC * N_PAD, FH), jnp.float32),
    mesh=_MESH,
    scratch_types=[
        pltpu.VMEM((NB, B), jnp.int32),
        pltpu.VMEM((NB, B), jnp.int32),
        pltpu.VMEM((NB, B), jnp.float32),
        pltpu.VMEM((B, FH), jnp.float32),
        pltpu.VMEM_SHARED((N_PAD, FH), jnp.float32),
        pltpu.SemaphoreType.DMA,
    ],
)
def _spmm(sup_hbm, cols_hbm, rows_hbm, vals_hbm, zeros_hbm, out_hbm,
          cols_v, rows_v, vals_v, gbuf, acc, sem):
    c = lax.axis_index("c")
    s = lax.axis_index("s")

    pltpu.sync_copy(cols_hbm.at[c, s], cols_v)
    pltpu.sync_copy(rows_hbm.at[s], rows_v)
    pltpu.sync_copy(vals_hbm.at[s], vals_v)
    pltpu.sync_copy(zeros_hbm, acc.at[pl.ds(s * RPT, RPT)])
    plsc.subcore_barrier()

    def batch_body(b, carry):
        pltpu.async_copy(sup_hbm.at[cols_v.at[b]], gbuf, sem).wait()

        def group_body(g, carry2):
            vv = vals_v[b, pl.ds(g * 16, 16)]
            for e in range(16):
                scale = _splat_lane(vv, e)
                row = g * 16 + e
                for f in range(FV):
                    sl = pl.ds(f * 16, 16)
                    gbuf[row, sl] = gbuf[row, sl] * scale
            return carry2

        lax.fori_loop(0, GROUPS, group_body, 0)
        pltpu.sync_copy(gbuf, acc.at[rows_v.at[b]], add=True)
        return carry

    lax.fori_loop(0, NB, batch_body, 0)
    plsc.subcore_barrier()

    base = c * N_PAD + s * RPT
    pltpu.sync_copy(acc.at[pl.ds(s * RPT, RPT)],
                    out_hbm.at[pl.ds(base, RPT)])


@jax.jit
def kernel(adj_rows, adj_cols, adj_vals, input_feature, weights):
    support = _support_halves(weights, input_feature)

    pad = E_PAD - E
    cols = jnp.concatenate(
        [adj_cols.astype(jnp.int32), jnp.zeros((pad,), jnp.int32)])
    rows = jnp.concatenate(
        [adj_rows.astype(jnp.int32), jnp.zeros((pad,), jnp.int32)])
    vals = jnp.concatenate([adj_vals, jnp.zeros((pad,), jnp.float32)])
    cols_r = cols.reshape(NS, NB, B)
    cols2 = jnp.stack([cols_r, cols_r + N])
    rows_r = rows.reshape(NS, NB, B)
    vals_r = vals.reshape(NS, NB, B)
    zeros = jnp.zeros((RPT, FH), jnp.float32)

    out2 = _spmm(support, cols2, rows_r, vals_r, zeros)
    halves = out2.reshape(NC, N_PAD, FH)[:, :N]
    return halves.transpose(1, 0, 2).reshape(N, F)
